# baseline (speedup 1.0000x reference)
import functools

import jax
import jax.numpy as jnp
from jax import lax
from jax.experimental import pallas as pl
from jax.experimental.pallas import tpu as pltpu
from jax.experimental.pallas import tpu_sc as plsc

_TOKENS = 1000000
_WORDS = 200
_COORDS = 32
_BATCH = 4096

_NW = 32
_ROWS = _BATCH * _WORDS
_ROWS_PER_W = _ROWS // _NW
_CHUNK = 512
_NCHUNK = _ROWS_PER_W // _CHUNK
_GATHER = 128
_NGATHER = _CHUNK // _GATHER
_MAX_PHASE = max((c * _CHUNK) % _WORDS for c in range(_NCHUNK))
_POS_ROWS = _MAX_PHASE + _CHUNK
_NBUF = 5
_LOOKAHEAD = 3


def _encoder(seq_flat, vocab_table, pos_table):
    mesh = plsc.VectorSubcoreMesh(core_axis_name="c", subcore_axis_name="s")

    @functools.partial(
        pl.kernel,
        mesh=mesh,
        out_type=jax.ShapeDtypeStruct((_ROWS, _COORDS), jnp.float32),
        scratch_types=[
            pltpu.VMEM((_ROWS_PER_W,), jnp.int32),
            pltpu.VMEM((_NBUF, _CHUNK, _COORDS), jnp.float32),
            pltpu.VMEM((_POS_ROWS, _COORDS), jnp.float32),
        ]
        + [pltpu.SemaphoreType.DMA] * (2 * _NBUF),
        compiler_params=pltpu.CompilerParams(use_tc_tiling_on_sc=False),
    )
    def body(seq_hbm, vocab_hbm, pos_hbm, out_hbm, idx_v, rows_v, pos_v,
             *sems):
        wid = lax.axis_index("s") * 2 + lax.axis_index("c")
        base = wid * _ROWS_PER_W
        gsems = sems[:_NBUF]
        osems = sems[_NBUF:]

        pltpu.sync_copy(seq_hbm.at[pl.ds(base, _ROWS_PER_W)], idx_v)

        for t in range(_POS_ROWS // _WORDS):
            pltpu.sync_copy(pos_hbm, pos_v.at[pl.ds(t * _WORDS, _WORDS)])
        rem = _POS_ROWS % _WORDS
        if rem:
            pltpu.sync_copy(
                pos_hbm.at[pl.ds(0, rem)],
                pos_v.at[pl.ds(_POS_ROWS - rem, rem)])

        def fire(c):
            buf = c % _NBUF
            return [
                pltpu.async_copy(
                    vocab_hbm.at[
                        idx_v.at[pl.ds(c * _CHUNK + j * _GATHER, _GATHER)]],
                    rows_v.at[buf, pl.ds(j * _GATHER, _GATHER)],
                    gsems[buf],
                )
                for j in range(_NGATHER)
            ]

        gathers = {c: fire(c) for c in range(min(_LOOKAHEAD, _NCHUNK))}
        writebacks = {}
        for c in range(_NCHUNK):
            buf = c % _NBUF
            nxt = c + _LOOKAHEAD
            if nxt < _NCHUNK:
                if nxt - _NBUF in writebacks:
                    writebacks.pop(nxt - _NBUF).wait()
                gathers[nxt] = fire(nxt)
            for cp in gathers.pop(c):
                cp.wait()

            p0 = (c * _CHUNK) % _WORDS

            @pl.loop(0, _CHUNK)
            def _(r):
                pr = r + p0
                v0 = pos_v[pr, pl.ds(0, 16)]
                v1 = pos_v[pr, pl.ds(16, 16)]
                plsc.addupdate(rows_v.at[buf, r, pl.ds(0, 16)], v0)
                plsc.addupdate(rows_v.at[buf, r, pl.ds(16, 16)], v1)

            writebacks[c] = pltpu.async_copy(
                rows_v.at[buf], out_hbm.at[pl.ds(base + c * _CHUNK, _CHUNK)],
                osems[buf],
            )
        for wb in writebacks.values():
            wb.wait()

    return body(seq_flat, vocab_table, pos_table)


def kernel(sequence_bw, vocab_table, pos_table):
    seq_flat = sequence_bw.reshape(-1).astype(jnp.int32)
    out = _encoder(seq_flat, vocab_table, pos_table)
    return out.reshape(_BATCH, _WORDS, _COORDS)


# speedup vs baseline: 1.0244x; 1.0244x over previous
import functools

import jax
import jax.numpy as jnp
from jax import lax
from jax.experimental import pallas as pl
from jax.experimental.pallas import tpu as pltpu
from jax.experimental.pallas import tpu_sc as plsc

_TOKENS = 1000000
_WORDS = 200
_COORDS = 32
_BATCH = 4096

_NW = 32
_SEQ_PER_W = _BATCH // _NW
_SPC = 2
_NCHUNK = _SEQ_PER_W // _SPC
_SPLITS = ((0, 96), (96, 104))
_NBUF = 5
_LOOKAHEAD = 3


def _encoder(seq_bw, vocab_table, pos_table):
    mesh = plsc.VectorSubcoreMesh(core_axis_name="c", subcore_axis_name="s")

    @functools.partial(
        pl.kernel,
        mesh=mesh,
        out_type=jax.ShapeDtypeStruct((_BATCH, _WORDS, _COORDS), jnp.float32),
        scratch_types=[
            pltpu.VMEM((_SEQ_PER_W, _WORDS), jnp.int32),
            pltpu.VMEM((_NBUF, _SPC, _WORDS, _COORDS), jnp.float32),
            pltpu.VMEM((_WORDS, _COORDS), jnp.float32),
        ]
        + [pltpu.SemaphoreType.DMA] * (2 * _NBUF),
        compiler_params=pltpu.CompilerParams(use_tc_tiling_on_sc=False),
    )
    def body(seq_hbm, vocab_hbm, pos_hbm, out_hbm, idx_v, rows_v, pos_v,
             *sems):
        wid = lax.axis_index("s") * 2 + lax.axis_index("c")
        base = wid * _SEQ_PER_W
        gsems = sems[:_NBUF]
        osems = sems[_NBUF:]

        pltpu.sync_copy(seq_hbm.at[pl.ds(base, _SEQ_PER_W)], idx_v)
        pltpu.sync_copy(pos_hbm, pos_v)

        def fire(c):
            buf = c % _NBUF
            return [
                pltpu.async_copy(
                    vocab_hbm.at[idx_v.at[c * _SPC + k, pl.ds(off, ln)]],
                    rows_v.at[buf, k, pl.ds(off, ln)],
                    gsems[buf],
                )
                for k in range(_SPC)
                for off, ln in _SPLITS
            ]

        gathers = {c: fire(c) for c in range(min(_LOOKAHEAD, _NCHUNK))}
        writebacks = {}
        for c in range(_NCHUNK):
            buf = c % _NBUF
            nxt = c + _LOOKAHEAD
            if nxt < _NCHUNK:
                if nxt - _NBUF in writebacks:
                    writebacks.pop(nxt - _NBUF).wait()
                gathers[nxt] = fire(nxt)
            for cp in gathers.pop(c):
                cp.wait()

            @pl.loop(0, _WORDS)
            def _(r):
                v0 = pos_v[r, pl.ds(0, 16)]
                v1 = pos_v[r, pl.ds(16, 16)]
                for k in range(_SPC):
                    plsc.addupdate(rows_v.at[buf, k, r, pl.ds(0, 16)], v0)
                    plsc.addupdate(rows_v.at[buf, k, r, pl.ds(16, 16)], v1)

            writebacks[c] = pltpu.async_copy(
                rows_v.at[buf], out_hbm.at[pl.ds(base + c * _SPC, _SPC)],
                osems[buf],
            )
        for wb in writebacks.values():
            wb.wait()

    return body(seq_bw, vocab_table, pos_table)


def kernel(sequence_bw, vocab_table, pos_table):
    return _encoder(sequence_bw.astype(jnp.int32), vocab_table, pos_table)
